# Initial kernel scaffold; baseline (speedup 1.0000x reference)
import jax
import jax.numpy as jnp
from jax.experimental import pallas as pl


def kernel(input_tensor, h_cur, c_cur, edge_index, W, att_src, att_dst, bias):
    raise NotImplementedError("write your pallas kernel here")


# speedup vs baseline: 5.7517x; 5.7517x over previous
import functools

import jax
import jax.numpy as jnp
from jax import lax
from jax.experimental import pallas as pl
from jax.experimental.pallas import tpu as pltpu
from jax.experimental.pallas import tpu_sc as plsc

B = 2
N = 10000
IN_DIM = 128
HID = 128
OUT = 4 * HID
E = 320000
E_TOT = E + N
NT = 16
K = 64
NB = 16
SB = NB * K
NSB = 21
PER_TILE = NSB * SB
E_PAD = NT * PER_TILE
N_PAD = 10240
N2 = 10240
NS = N2 // NT
DS = N_PAD // NT
ZR = 32
RB = 1000


def _mm_body(x_ref, hc_ref, wx_ref, wh_ref, as_ref, ad_ref,
             h0, h1, h2, h3, asrc_o, adst_o):
    for b in range(B):
        h = (jnp.dot(x_ref[b], wx_ref[...], preferred_element_type=jnp.float32)
             + jnp.dot(hc_ref[b], wh_ref[...],
                       preferred_element_type=jnp.float32))
        asrc_o[:, b] = jnp.sum(h * as_ref[...], axis=1)
        adst_o[:, b] = jnp.sum(h * ad_ref[...], axis=1)
        h0[b] = h[:, 0 * HID:1 * HID]
        h1[b] = h[:, 1 * HID:2 * HID]
        h2[b] = h[:, 2 * HID:3 * HID]
        h3[b] = h[:, 3 * HID:4 * HID]


def _gat_matmul(x, hc, Wx, Wh, att_s_row, att_d_row):
    grid = (N // RB,)
    blk3 = pl.BlockSpec((B, RB, HID), lambda n: (0, n, 0))
    blk2 = pl.BlockSpec((RB, B), lambda n: (n, 0))
    full = lambda shape: pl.BlockSpec(shape, lambda n: tuple(0 for _ in shape))
    return pl.pallas_call(
        _mm_body,
        grid=grid,
        in_specs=[
            blk3, blk3,
            full((IN_DIM, OUT)), full((HID, OUT)),
            full((1, OUT)), full((1, OUT)),
        ],
        out_specs=[blk3, blk3, blk3, blk3, blk2, blk2],
        out_shape=[jax.ShapeDtypeStruct((B, N, HID), jnp.float32)] * 4
        + [jax.ShapeDtypeStruct((N, B), jnp.float32)] * 2,
    )(x, hc, Wx, Wh, att_s_row, att_d_row)


def _sc_body(h0, h1, h2, h3, asrc_hbm, adst_hbm, srci_hbm, dsti_hbm,
             num0, num1, num2, num3, den_out, w_hbm,
             src_sb, dst_sb, w_sb, asrc_v, adst_v, dpart_v, rows_v,
             tmp_v, acc_sh, sem):
    bidx = lax.axis_index("c")
    tid = lax.axis_index("s")
    iota = lax.iota(jnp.int32, 16)
    zero16 = jnp.zeros((16,), jnp.float32)
    neg16 = jnp.full((16,), -1e30, jnp.float32)

    pltpu.sync_copy(asrc_hbm.at[pl.ds(bidx * N, N)], asrc_v)
    pltpu.sync_copy(adst_hbm.at[pl.ds(bidx * N, N)], adst_v)

    ms = lax.fori_loop(
        0, N // 16,
        lambda i, m: jnp.maximum(m, asrc_v[pl.ds(i * 16, 16)]), neg16)
    md = lax.fori_loop(
        0, N // 16,
        lambda i, m: jnp.maximum(m, adst_v[pl.ds(i * 16, 16)]), neg16)

    def _lanemax(v):
        for k in (8, 4, 2, 1):
            tmp_v[pl.ds(0, 16)] = v
            v = jnp.maximum(v, plsc.load_gather(tmp_v, [iota ^ k]))
        return v
    M = jnp.maximum(_lanemax(ms) + _lanemax(md), 0.0)

    def _zero(i, _):
        r16 = jnp.broadcast_to(i >> 3, (16,))
        c16 = iota + (i & 7) * 16
        plsc.store_scatter(dpart_v, [r16, c16], zero16)
        return 0
    lax.fori_loop(0, N_PAD // 16, _zero, 0)

    ebase = tid * PER_TILE

    def _p1(sb, _):
        pltpu.sync_copy(srci_hbm.at[tid, sb], src_sb)
        pltpu.sync_copy(dsti_hbm.at[tid, sb], dst_sb)

        def _grp(g, _):
            r16 = jnp.broadcast_to(g >> 2, (16,))
            col = iota + (g & 3) * 16
            s16 = plsc.load_gather(src_sb, [r16, col])
            d16 = plsc.load_gather(dst_sb, [r16, col])
            a = plsc.load_gather(asrc_v, [s16]) + plsc.load_gather(adst_v, [d16])
            a = jnp.where(a >= 0.0, a, 0.2 * a)
            w = jnp.exp(a - M)
            eid = ebase + sb * SB + g * 16 + iota
            w = jnp.where(eid < E_TOT, w, 0.0)
            plsc.store_scatter(w_sb, [r16, col], w)
            plsc.addupdate_scatter(
                dpart_v, [lax.shift_right_logical(d16, 7), d16 & 127], w)
            return 0
        lax.fori_loop(0, SB // 16, _grp, 0)
        pltpu.sync_copy(w_sb, w_hbm.at[bidx, tid, sb])
        return 0
    lax.fori_loop(0, NSB, _p1, 0)

    PR = N_PAD // HID
    RR = DS // HID
    pltpu.sync_copy(dpart_v, acc_sh.at[pl.ds(tid * PR, PR)])
    plsc.subcore_barrier()

    def _zt(i, _):
        tmp_v[pl.ds(i * 16, 16)] = zero16
        return 0
    lax.fori_loop(0, DS // 16, _zt, 0)
    for p in range(NT):
        pltpu.sync_copy(acc_sh.at[pl.ds(p * PR + tid * RR, RR)],
                        rows_v.at[pl.ds(0, RR)])
        for r in range(RR):
            for c in range(HID // 16):
                q = r * (HID // 16) + c
                tmp_v[pl.ds(q * 16, 16)] = (tmp_v[pl.ds(q * 16, 16)]
                                            + rows_v[r, pl.ds(c * 16, 16)])
    pltpu.sync_copy(tmp_v, den_out.at[pl.ds(bidx * N_PAD + tid * DS, DS)])
    plsc.subcore_barrier()

    for g, (hg, numg) in enumerate(((h0, num0), (h1, num1), (h2, num2),
                                    (h3, num3))):
        def _zr(i, _):
            r16 = jnp.broadcast_to(i >> 3, (16,))
            c16 = iota + (i & 7) * 16
            plsc.store_scatter(rows_v, [r16, c16], zero16)
            return 0
        lax.fori_loop(0, (K * HID) // 16, _zr, 0)
        for z in range(NS // K):
            pltpu.sync_copy(rows_v, acc_sh.at[pl.ds(tid * NS + z * K, K)])
        plsc.subcore_barrier()

        def _p2(sb, _):
            pltpu.sync_copy(srci_hbm.at[tid, sb], src_sb)
            pltpu.sync_copy(dsti_hbm.at[tid, sb], dst_sb)
            pltpu.sync_copy(w_hbm.at[bidx, tid, sb], w_sb)

            def _blk(kb, _):
                pltpu.async_copy(hg.at[bidx].at[src_sb.at[kb]], rows_v,
                                 sem).wait()

                def _pe(e, _):
                    e16 = jnp.broadcast_to(e, (16,))
                    kb16 = jnp.broadcast_to(kb, (16,))
                    w16 = plsc.load_gather(w_sb, [kb16, e16])
                    for j in range(HID // 16):
                        cj = iota + j * 16
                        v = plsc.load_gather(rows_v, [e16, cj])
                        plsc.store_scatter(rows_v, [e16, cj], v * w16)
                    return 0
                lax.fori_loop(0, K, _pe, 0)
                pltpu.sync_copy(rows_v, acc_sh.at[dst_sb.at[kb]], add=True)
                return 0
            lax.fori_loop(0, NB, _blk, 0)
            return 0
        lax.fori_loop(0, NSB, _p2, 0)
        plsc.subcore_barrier()
        pltpu.sync_copy(acc_sh.at[pl.ds(tid * NS, NS)],
                        numg.at[bidx, pl.ds(tid * NS, NS)])


def _sc_gat(h0, h1, h2, h3, asrc, adst, srci, dsti):
    mesh = plsc.VectorSubcoreMesh(core_axis_name="c", subcore_axis_name="s")
    run = pl.kernel(
        _sc_body,
        out_type=[jax.ShapeDtypeStruct((B, N2, HID), jnp.float32)] * 4
        + [jax.ShapeDtypeStruct((B * N_PAD,), jnp.float32),
           jax.ShapeDtypeStruct((B, NT, NSB, NB, K), jnp.float32)],
        mesh=mesh,
        compiler_params=pltpu.CompilerParams(needs_layout_passes=False),
        scratch_types=[
            pltpu.VMEM((NB, K), jnp.int32),
            pltpu.VMEM((NB, K), jnp.int32),
            pltpu.VMEM((NB, K), jnp.float32),
            pltpu.VMEM((N,), jnp.float32),
            pltpu.VMEM((N,), jnp.float32),
            pltpu.VMEM((N_PAD // HID, HID), jnp.float32),
            pltpu.VMEM((K, HID), jnp.float32),
            pltpu.VMEM((DS,), jnp.float32),
            pltpu.VMEM_SHARED((N2, HID), jnp.float32),
            pltpu.SemaphoreType.DMA,
        ],
    )
    return run(h0, h1, h2, h3, asrc, adst, srci, dsti)


def _lstm_body(n0, n1, n2, n3, den_ref, c_ref, bias_ref, h_out, c_out):
    for b in range(B):
        inv = (1.0 / (den_ref[:, b] + 1e-30))[:, None]
        i = jax.nn.sigmoid(n0[b] * inv + bias_ref[0])
        f = jax.nn.sigmoid(n1[b] * inv + bias_ref[1])
        o = jax.nn.sigmoid(n2[b] * inv + bias_ref[2])
        g = jnp.tanh(n3[b] * inv + bias_ref[3])
        c_next = f * c_ref[b] + i * g
        h_out[b] = o * jnp.tanh(c_next)
        c_out[b] = c_next


def _lstm(num0, num1, num2, num3, den, c_cur, bias4):
    grid = (N // RB,)
    blk3 = pl.BlockSpec((B, RB, HID), lambda n: (0, n, 0))
    blk2 = pl.BlockSpec((RB, B), lambda n: (n, 0))
    return pl.pallas_call(
        _lstm_body,
        grid=grid,
        in_specs=[blk3, blk3, blk3, blk3, blk2, blk3,
                  pl.BlockSpec((4, HID), lambda n: (0, 0))],
        out_specs=[blk3, blk3],
        out_shape=[jax.ShapeDtypeStruct((B, N, HID), jnp.float32)] * 2,
    )(num0, num1, num2, num3, den, c_cur, bias4)


def kernel(input_tensor, h_cur, c_cur, edge_index, W, att_src, att_dst, bias):
    Wx = W[:IN_DIM]
    Wh = W[IN_DIM:]
    att_s_row = att_src.reshape(1, OUT)
    att_d_row = att_dst.reshape(1, OUT)
    bias4 = bias.reshape(4, HID)

    loop = jnp.arange(N, dtype=edge_index.dtype)
    pad = jnp.zeros((E_PAD - E_TOT,), dtype=edge_index.dtype)
    src = jnp.concatenate([edge_index[0], loop, pad]).reshape(NT, NSB, NB, K)
    dst = jnp.concatenate([edge_index[1], loop, pad]).reshape(NT, NSB, NB, K)

    h0, h1, h2, h3, asrc_t, adst_t = _gat_matmul(
        input_tensor, h_cur, Wx, Wh, att_s_row, att_d_row)
    num0, num1, num2, num3, den, _w = _sc_gat(
        h0, h1, h2, h3, asrc_t.T.reshape(-1), adst_t.T.reshape(-1), src, dst)
    den_t = den.reshape(B, N_PAD).T
    return _lstm(num0, num1, num2, num3, den_t, c_cur, bias4)


# speedup vs baseline: 11.6345x; 2.0228x over previous
import jax
import jax.numpy as jnp
from jax import lax
from jax.experimental import pallas as pl
from jax.experimental.pallas import tpu as pltpu
from jax.experimental.pallas import tpu_sc as plsc

B = 2
N = 10000
IN_DIM = 128
HID = 128
OUT = 4 * HID
E = 320000
E_TOT = E + N
NT = 16
K = 16
NB = 64
SB = NB * K
NSB = 21
PER_TILE = NSB * SB
E_PAD = NT * PER_TILE
N_PAD = 10240
N2 = 10240
NS = N2 // NT
DS = N_PAD // NT
PR = N_PAD // HID
RR = DS // HID
RB = 1000
_SC_PARAMS = pltpu.CompilerParams(needs_layout_passes=False)
_MESH = dict(core_axis_name="c", subcore_axis_name="s")


def _mm_body(x_ref, hc_ref, wx_ref, wh_ref, as_ref, ad_ref,
             h0, h1, h2, h3, asrc_o, adst_o):
    hs = (h0, h1, h2, h3)
    for b in range(B):
        h = (jnp.dot(x_ref[b], wx_ref[...], preferred_element_type=jnp.float32)
             + jnp.dot(hc_ref[b], wh_ref[...],
                       preferred_element_type=jnp.float32))
        asrc_o[:, b] = jnp.sum(h * as_ref[...], axis=1)
        adst_o[:, b] = jnp.sum(h * ad_ref[...], axis=1)
        for g in range(4):
            hs[g][b] = h[:, g * HID:(g + 1) * HID]


def _gat_matmul(x, hc, Wx, Wh, att_s_row, att_d_row):
    grid = (N // RB,)
    blk3 = pl.BlockSpec((B, RB, HID), lambda n: (0, n, 0))
    blk2 = pl.BlockSpec((RB, B), lambda n: (n, 0))
    full = lambda shape: pl.BlockSpec(shape, lambda n: tuple(0 for _ in shape))
    return pl.pallas_call(
        _mm_body,
        grid=grid,
        in_specs=[
            pl.BlockSpec((B, RB, IN_DIM), lambda n: (0, n, 0)),
            blk3,
            full((IN_DIM, OUT)), full((HID, OUT)),
            full((1, OUT)), full((1, OUT)),
        ],
        out_specs=[blk3] * 4 + [blk2, blk2],
        out_shape=[jax.ShapeDtypeStruct((B, N, HID), jnp.float32)] * 4
        + [jax.ShapeDtypeStruct((N, B), jnp.float32)] * 2,
    )(x, hc, Wx, Wh, att_s_row, att_d_row)


def _w_body(asrc_hbm, adst_hbm, srci_hbm, dsti_hbm,
            den_out, w_hbm,
            src_sb, dst_sb, w_sb, asrc_v, adst_v, dpart_v, tmp_v, ld_v,
            dpart_sh):
    bidx = lax.axis_index("c")
    tid = lax.axis_index("s")
    iota = lax.iota(jnp.int32, 16)
    zero16 = jnp.zeros((16,), jnp.float32)
    neg16 = jnp.full((16,), -1e30, jnp.float32)

    pltpu.sync_copy(asrc_hbm.at[pl.ds(bidx * N, N)], asrc_v)
    pltpu.sync_copy(adst_hbm.at[pl.ds(bidx * N, N)], adst_v)

    ms = lax.fori_loop(
        0, N // 16,
        lambda i, m: jnp.maximum(m, asrc_v[pl.ds(i * 16, 16)]), neg16)
    md = lax.fori_loop(
        0, N // 16,
        lambda i, m: jnp.maximum(m, adst_v[pl.ds(i * 16, 16)]), neg16)

    def _lanemax(v):
        for k in (8, 4, 2, 1):
            tmp_v[pl.ds(0, 16)] = v
            v = jnp.maximum(v, plsc.load_gather(tmp_v, [iota ^ k]))
        return v
    M = jnp.maximum(_lanemax(ms) + _lanemax(md), 0.0)

    def _zero(i, _):
        r16 = jnp.broadcast_to(i >> 3, (16,))
        c16 = iota + (i & 7) * 16
        plsc.store_scatter(dpart_v, [r16, c16], zero16)
        return 0
    lax.fori_loop(0, N_PAD // 16, _zero, 0)

    ebase = tid * PER_TILE

    def _p1(sb, _):
        pltpu.sync_copy(srci_hbm.at[tid, sb], src_sb)
        pltpu.sync_copy(dsti_hbm.at[tid, sb], dst_sb)

        def _grp(gi, _):
            r16 = jnp.broadcast_to(gi, (16,))
            s16 = plsc.load_gather(src_sb, [r16, iota])
            d16 = plsc.load_gather(dst_sb, [r16, iota])
            a = plsc.load_gather(asrc_v, [s16]) + plsc.load_gather(adst_v, [d16])
            a = jnp.where(a >= 0.0, a, 0.2 * a)
            w = jnp.exp(a - M)
            eid = ebase + sb * SB + gi * 16 + iota
            w = jnp.where(eid < E_TOT, w, 0.0)
            plsc.store_scatter(w_sb, [r16, iota], w)
            plsc.addupdate_scatter(
                dpart_v, [lax.shift_right_logical(d16, 7), d16 & 127], w)
            return 0
        lax.fori_loop(0, NB, _grp, 0)
        pltpu.sync_copy(w_sb, w_hbm.at[bidx, tid, sb])
        return 0
    lax.fori_loop(0, NSB, _p1, 0)

    pltpu.sync_copy(dpart_v, dpart_sh.at[pl.ds(tid * PR, PR)])
    plsc.subcore_barrier()

    def _zt(i, _):
        tmp_v[pl.ds(i * 16, 16)] = zero16
        return 0
    lax.fori_loop(0, DS // 16, _zt, 0)
    for p in range(NT):
        pltpu.sync_copy(dpart_sh.at[pl.ds(p * PR + tid * RR, RR)], ld_v)
        for r in range(RR):
            for c in range(HID // 16):
                q = r * (HID // 16) + c
                tmp_v[pl.ds(q * 16, 16)] = (tmp_v[pl.ds(q * 16, 16)]
                                            + ld_v[r, pl.ds(c * 16, 16)])
    pltpu.sync_copy(tmp_v, den_out.at[pl.ds(bidx * N_PAD + tid * DS, DS)])


def _sc_weights(asrc, adst, srci, dsti):
    run = pl.kernel(
        _w_body,
        out_type=[jax.ShapeDtypeStruct((B * N_PAD,), jnp.float32),
                  jax.ShapeDtypeStruct((B, NT, NSB, NB, K), jnp.float32)],
        mesh=plsc.VectorSubcoreMesh(**_MESH),
        compiler_params=_SC_PARAMS,
        scratch_types=[
            pltpu.VMEM((NB, K), jnp.int32),
            pltpu.VMEM((NB, K), jnp.int32),
            pltpu.VMEM((NB, K), jnp.float32),
            pltpu.VMEM((N,), jnp.float32),
            pltpu.VMEM((N,), jnp.float32),
            pltpu.VMEM((PR, HID), jnp.float32),
            pltpu.VMEM((DS,), jnp.float32),
            pltpu.VMEM((RR, HID), jnp.float32),
            pltpu.VMEM_SHARED((NT * PR, HID), jnp.float32),
        ],
    )
    return run(asrc, adst, srci, dsti)


def _scatter_body(hg_hbm, srci_hbm, dsti_hbm, w_hbm, numg,
                  src_sb, dst_sb, w_sb, buf0, buf1, buf2, buf3, acc_sh,
                  gs0, gs1, gs2, gs3, ss0, ss1, ss2, ss3):
    bidx = lax.axis_index("c")
    tid = lax.axis_index("s")
    iota = lax.iota(jnp.int32, 16)
    zero16 = jnp.zeros((16,), jnp.float32)
    bufs = (buf0, buf1, buf2, buf3)
    gsems = (gs0, gs1, gs2, gs3)
    ssems = (ss0, ss1, ss2, ss3)
    NBQ = NB // 4
    hgb = hg_hbm.at[bidx]

    def _zr(i, _):
        r16 = jnp.broadcast_to(i >> 3, (16,))
        c16 = iota + (i & 7) * 16
        plsc.store_scatter(buf0, [r16, c16], zero16)
        return 0
    lax.fori_loop(0, (K * HID) // 16, _zr, 0)
    for z in range(NS // K):
        pltpu.sync_copy(buf0, acc_sh.at[pl.ds(tid * NS + z * K, K)])
    plsc.subcore_barrier()

    def _p2(sb, _):
        pltpu.sync_copy(srci_hbm.at[tid, sb], src_sb)
        pltpu.sync_copy(dsti_hbm.at[tid, sb], dst_sb)
        pltpu.sync_copy(w_hbm.at[bidx, tid, sb], w_sb)

        def gstart(p, kb):
            pltpu.async_copy(hgb.at[src_sb.at[kb]], bufs[p], gsems[p])

        def gwait(p, kb):
            pltpu.make_async_copy(hgb.at[src_sb.at[kb]], bufs[p],
                                  gsems[p]).wait()

        def sstart(p, kb):
            pltpu.async_copy(bufs[p], acc_sh.at[dst_sb.at[kb]], ssems[p],
                             add=True)

        def swait(p):
            pltpu.make_async_copy(bufs[p], acc_sh.at[dst_sb.at[0]],
                                  ssems[p]).wait()

        gstart(0, 0)
        gstart(1, 1)

        def _i4(i4, _):
            for par in range(4):
                kb = i4 * 4 + par
                p2 = (par + 2) & 3
                if par < 2:
                    @pl.when(i4 > 0)
                    def _():
                        swait(p2)
                    gstart(p2, kb + 2)
                else:
                    @pl.when(i4 < NBQ - 1)
                    def _():
                        swait(p2)
                        gstart(p2, kb + 2)
                gwait(par, kb)
                kb16 = jnp.broadcast_to(kb, (16,))
                buf = bufs[par]
                for e in range(K):
                    we = plsc.load_gather(
                        w_sb, [kb16, jnp.full((16,), e, jnp.int32)])
                    for j in range(HID // 16):
                        sl = pl.ds(j * 16, 16)
                        buf[e, sl] = buf[e, sl] * we
                sstart(par, kb)
            return 0
        lax.fori_loop(0, NBQ, _i4, 0)
        for p in range(4):
            swait(p)
        return 0
    lax.fori_loop(0, NSB, _p2, 0)
    plsc.subcore_barrier()
    pltpu.sync_copy(acc_sh.at[pl.ds(tid * NS, NS)],
                    numg.at[bidx, pl.ds(tid * NS, NS)])


def _sc_scatter(hg, srci, dsti, w):
    run = pl.kernel(
        _scatter_body,
        out_type=jax.ShapeDtypeStruct((B, N2, HID), jnp.float32),
        mesh=plsc.VectorSubcoreMesh(**_MESH),
        compiler_params=_SC_PARAMS,
        scratch_types=[
            pltpu.VMEM((NB, K), jnp.int32),
            pltpu.VMEM((NB, K), jnp.int32),
            pltpu.VMEM((NB, K), jnp.float32),
            pltpu.VMEM((K, HID), jnp.float32),
            pltpu.VMEM((K, HID), jnp.float32),
            pltpu.VMEM((K, HID), jnp.float32),
            pltpu.VMEM((K, HID), jnp.float32),
            pltpu.VMEM_SHARED((N2, HID), jnp.float32),
            pltpu.SemaphoreType.DMA, pltpu.SemaphoreType.DMA,
            pltpu.SemaphoreType.DMA, pltpu.SemaphoreType.DMA,
            pltpu.SemaphoreType.DMA, pltpu.SemaphoreType.DMA,
            pltpu.SemaphoreType.DMA, pltpu.SemaphoreType.DMA,
        ],
    )
    return run(hg, srci, dsti, w)


def _lstm_body(n0, n1, n2, n3, den_ref, c_ref, bias_ref, h_out, c_out):
    nums = (n0, n1, n2, n3)
    for b in range(B):
        inv = (1.0 / (den_ref[:, b] + 1e-30))[:, None]
        i = jax.nn.sigmoid(nums[0][b] * inv + bias_ref[0])
        f = jax.nn.sigmoid(nums[1][b] * inv + bias_ref[1])
        o = jax.nn.sigmoid(nums[2][b] * inv + bias_ref[2])
        gg = jnp.tanh(nums[3][b] * inv + bias_ref[3])
        c_next = f * c_ref[b] + i * gg
        h_out[b] = o * jnp.tanh(c_next)
        c_out[b] = c_next


def _lstm(nums, den, c_cur, bias4):
    grid = (N // RB,)
    blk3 = pl.BlockSpec((B, RB, HID), lambda n: (0, n, 0))
    blk2 = pl.BlockSpec((RB, B), lambda n: (n, 0))
    return pl.pallas_call(
        _lstm_body,
        grid=grid,
        in_specs=[blk3, blk3, blk3, blk3, blk2, blk3,
                  pl.BlockSpec((4, HID), lambda n: (0, 0))],
        out_specs=[blk3, blk3],
        out_shape=[jax.ShapeDtypeStruct((B, N, HID), jnp.float32)] * 2,
    )(*nums, den, c_cur, bias4)


def kernel(input_tensor, h_cur, c_cur, edge_index, W, att_src, att_dst, bias):
    Wx = W[:IN_DIM]
    Wh = W[IN_DIM:]
    att_s_row = att_src.reshape(1, OUT)
    att_d_row = att_dst.reshape(1, OUT)
    bias4 = bias.reshape(4, HID)

    loop = jnp.arange(N, dtype=edge_index.dtype)
    pad = jnp.zeros((E_PAD - E_TOT,), dtype=edge_index.dtype)
    src = jnp.concatenate([edge_index[0], loop, pad]).reshape(NT, NSB, NB, K)
    dst = jnp.concatenate([edge_index[1], loop, pad]).reshape(NT, NSB, NB, K)

    h0, h1, h2, h3, asrc_t, adst_t = _gat_matmul(
        input_tensor, h_cur, Wx, Wh, att_s_row, att_d_row)
    den, w = _sc_weights(asrc_t.T.reshape(-1), adst_t.T.reshape(-1), src, dst)
    nums = [_sc_scatter(hg, src, dst, w) for hg in (h0, h1, h2, h3)]
    den_t = den.reshape(B, N_PAD).T
    return _lstm(nums, den_t, c_cur, bias4)
